# Initial kernel scaffold; baseline (speedup 1.0000x reference)
import jax
import jax.numpy as jnp
from jax.experimental import pallas as pl


def kernel(x, edge_index, W_init, W1, b1, W2, b2, W3, b3):
    raise NotImplementedError("write your pallas kernel here")


# speedup vs baseline: 8.7522x; 8.7522x over previous
import functools

import jax
import jax.numpy as jnp
from jax import lax
from jax.experimental import pallas as pl
from jax.experimental.pallas import tpu as pltpu
from jax.experimental.pallas import tpu_sc as plsc

N = 10000
NPAD = 10240
E = 320000
H = 128

NC = 2
NS = 16
NW = NC * NS
EPW = E // NW
C = 80
NCHUNK = EPW // C
RPT = NPAD // NS

_mesh = plsc.VectorSubcoreMesh(core_axis_name="c", subcore_axis_name="s",
                               num_cores=NC, num_subcores=NS)


def _deg_body(dst_hbm, out_hbm, dst_v, ones_v, zbuf, acc):
    cid = lax.axis_index("c")
    sid = lax.axis_index("s")
    wid = sid * NC + cid

    zero16 = jnp.zeros((16,), jnp.float32)
    one16 = jnp.ones((16,), jnp.float32)

    def fill(r, _):
        for c in range(H // 16):
            zbuf[r, pl.ds(c * 16, 16)] = zero16
        return 0

    lax.fori_loop(0, 128, fill, 0)

    def fill1(r, _):
        for c in range(H // 16):
            ones_v[r, pl.ds(c * 16, 16)] = one16
        return 0

    lax.fori_loop(0, C, fill1, 0)

    for k in range(RPT // 128):
        pltpu.sync_copy(zbuf, acc.at[pl.ds(sid * RPT + k * 128, 128)])
    plsc.subcore_barrier()

    ebase = wid * EPW

    def body(i, _):
        pltpu.sync_copy(dst_hbm.at[pl.ds(ebase + i * C, C)], dst_v)
        pltpu.sync_copy(ones_v, acc.at[dst_v], add=True)
        return 0

    lax.fori_loop(0, NCHUNK, body, 0)
    plsc.subcore_barrier()
    pltpu.sync_copy(acc.at[pl.ds(sid * RPT, RPT)],
                    out_hbm.at[cid, pl.ds(sid * RPT, RPT)])


def _agg_body(g_hbm, src_hbm, dst_hbm, out_hbm,
              src_v, dst_v, rows_v, zbuf, acc, sem):
    cid = lax.axis_index("c")
    sid = lax.axis_index("s")
    wid = sid * NC + cid

    zero16 = jnp.zeros((16,), jnp.float32)

    def fill(r, _):
        for c in range(H // 16):
            zbuf[r, pl.ds(c * 16, 16)] = zero16
        return 0

    lax.fori_loop(0, 128, fill, 0)
    for k in range(RPT // 128):
        pltpu.sync_copy(zbuf, acc.at[pl.ds(sid * RPT + k * 128, 128)])
    plsc.subcore_barrier()

    ebase = wid * EPW

    def body(i, _):
        off = ebase + i * C
        pltpu.sync_copy(src_hbm.at[pl.ds(off, C)], src_v)
        pltpu.sync_copy(dst_hbm.at[pl.ds(off, C)], dst_v)
        pltpu.async_copy(g_hbm.at[src_v], rows_v, sem).wait()
        pltpu.sync_copy(rows_v, acc.at[dst_v], add=True)
        return 0

    lax.fori_loop(0, NCHUNK, body, 0)
    plsc.subcore_barrier()
    pltpu.sync_copy(acc.at[pl.ds(sid * RPT, RPT)],
                    out_hbm.at[cid, pl.ds(sid * RPT, RPT)])


def _make_deg_kernel(interpret=False):
    return pl.kernel(
        _deg_body,
        out_type=jax.ShapeDtypeStruct((NC, NPAD, H), jnp.float32),
        mesh=_mesh,
        scratch_types=[
            pltpu.VMEM((C,), jnp.int32),
            pltpu.VMEM((C, H), jnp.float32),
            pltpu.VMEM((128, H), jnp.float32),
            pltpu.VMEM_SHARED((NPAD, H), jnp.float32),
        ],
        interpret=interpret,
    )


def _make_agg_kernel(interpret=False):
    return pl.kernel(
        _agg_body,
        out_type=jax.ShapeDtypeStruct((NC, NPAD, H), jnp.float32),
        mesh=_mesh,
        scratch_types=[
            pltpu.VMEM((C,), jnp.int32),
            pltpu.VMEM((C,), jnp.int32),
            pltpu.VMEM((C, H), jnp.float32),
            pltpu.VMEM((128, H), jnp.float32),
            pltpu.VMEM_SHARED((NPAD, H), jnp.float32),
            pltpu.SemaphoreType.DMA,
        ],
        interpret=interpret,
    )


_deg_kernel = _make_deg_kernel()
_agg_kernel = _make_agg_kernel()


RB = 1000


def _tc0_body(deg_ref, x_ref, w_ref, g_ref, dinv_ref):
    deg = deg_ref[0, :, 0] + deg_ref[1, :, 0]
    dinv = lax.rsqrt(jnp.maximum(deg, 1.0))
    h0 = jnp.dot(x_ref[...], w_ref[...], preferred_element_type=jnp.float32)
    g_ref[...] = h0 * dinv[:, None]
    dinv_ref[...] = dinv[:, None]


def _tc0(deg, x, w):
    return pl.pallas_call(
        _tc0_body,
        grid=(N // RB,),
        in_specs=[
            pl.BlockSpec((NC, RB, H), lambda i: (0, i, 0)),
            pl.BlockSpec((RB, H), lambda i: (i, 0)),
            pl.BlockSpec((H, H), lambda i: (0, 0)),
        ],
        out_specs=[
            pl.BlockSpec((RB, H), lambda i: (i, 0)),
            pl.BlockSpec((RB, 1), lambda i: (i, 0)),
        ],
        out_shape=[
            jax.ShapeDtypeStruct((N, H), jnp.float32),
            jax.ShapeDtypeStruct((N, 1), jnp.float32),
        ],
    )(deg, x, w)


def _tc_layer_body(scale_out, p_ref, dinv_ref, w_ref, b_ref, out_ref):
    t = (p_ref[0] + p_ref[1]) * dinv_ref[...]
    h = jnp.dot(t, w_ref[...], preferred_element_type=jnp.float32)
    h = jnp.maximum(h + b_ref[...], 0.0)
    if scale_out:
        h = h * dinv_ref[...]
    out_ref[...] = h


def _tc_layer(p, dinv, w, b, scale_out):
    return pl.pallas_call(
        functools.partial(_tc_layer_body, scale_out),
        grid=(N // RB,),
        in_specs=[
            pl.BlockSpec((NC, RB, H), lambda i: (0, i, 0)),
            pl.BlockSpec((RB, 1), lambda i: (i, 0)),
            pl.BlockSpec((H, H), lambda i: (0, 0)),
            pl.BlockSpec((1, H), lambda i: (0, 0)),
        ],
        out_specs=pl.BlockSpec((RB, H), lambda i: (i, 0)),
        out_shape=jax.ShapeDtypeStruct((N, H), jnp.float32),
    )(p, dinv, w, b)


def kernel(x, edge_index, W_init, W1, b1, W2, b2, W3, b3):
    src = edge_index[0]
    dst = edge_index[1]
    degp = _deg_kernel(dst)
    g, dinv = _tc0(degp, x, W_init)
    for w, b, scale_out in ((W1, b1, True), (W2, b2, True), (W3, b3, False)):
        p = _agg_kernel(g, src, dst)
        g = _tc_layer(p, dinv, w, b.reshape(1, H), scale_out)
    return g
